# baseline (speedup 1.0000x reference)
import jax
import jax.numpy as jnp
from jax import lax
from jax.experimental import pallas as pl
from jax.experimental.pallas import tpu as pltpu
from jax.experimental.pallas import tpu_sc as plsc

N = 100000
E = 6400000
NC = 2
NS = 16
L = 16
NW = NC * NS
N_PAD = 100352
SLICE = N_PAD // NS
E_PER_TILE = E // NW
CHUNK = 1600
N_CHUNKS = E_PER_TILE // CHUNK
ROWS = N_PAD // 128

_mesh = plsc.VectorSubcoreMesh(
    core_axis_name="c", subcore_axis_name="s", num_cores=NC, num_subcores=NS
)


def _make_edge_pass(with_deg, depth, chunk=CHUNK):
    out_type = [jax.ShapeDtypeStruct((NC * N_PAD,), jnp.float32)]
    scratch = (
        [pltpu.VMEM((N_PAD,), jnp.float32)]
        + [pltpu.VMEM((chunk,), jnp.int32)] * 2
        + [pltpu.VMEM((chunk,), jnp.int32)] * depth
        + [pltpu.VMEM((chunk,), jnp.float32)] * depth
        + [
            pltpu.VMEM_SHARED((N_PAD,), jnp.float32),
            pltpu.SemaphoreType.DMA((2,)),
            pltpu.SemaphoreType.DMA((depth,)),
            pltpu.SemaphoreType.DMA((depth,)),
        ]
    )
    if with_deg:
        out_type.append(jax.ShapeDtypeStruct((NC * N_PAD,), jnp.float32))
        scratch.append(pltpu.VMEM((chunk,), jnp.float32))
        scratch.append(pltpu.VMEM_SHARED((N_PAD,), jnp.float32))
        scratch.append(pltpu.SemaphoreType.DMA((depth,)))

    n_chunks = E_PER_TILE // chunk
    grp = 2 * depth if depth % 2 else depth
    total_iters = n_chunks + depth - 1
    n_groups = (total_iters + grp - 1) // grp

    def body(values_hbm, src_hbm, dst_hbm, s_out, *rest):
        rest = list(rest)
        if with_deg:
            deg_out = rest.pop(0)
        table = rest.pop(0)
        srcb = [rest.pop(0) for _ in range(2)]
        dstb = [rest.pop(0) for _ in range(depth)]
        valb = [rest.pop(0) for _ in range(depth)]
        s_sh, es, ed, sc = rest[:4]
        if with_deg:
            onesb, deg_sh, dg = rest[4:]
        c = lax.axis_index("c")
        s = lax.axis_index("s")
        wid = s * NC + c
        ebase = wid * E_PER_TILE

        pltpu.async_copy(src_hbm.at[pl.ds(ebase, chunk)], srcb[0],
                         es.at[0])
        pltpu.async_copy(dst_hbm.at[pl.ds(ebase, chunk)], dstb[0],
                         ed.at[0])

        pltpu.sync_copy(values_hbm, table.at[pl.ds(0, values_hbm.shape[0])])

        zb = valb[depth - 1]

        def zloop(i, carry):
            zb[pl.ds(i * L, L)] = jnp.zeros((L,), jnp.float32)
            return carry
        lax.fori_loop(0, chunk // L, zloop, 0)
        zdsts = [s_sh] + ([deg_sh] if with_deg else [])
        for zdst in zdsts:
            for k in range(SLICE // chunk):
                pltpu.sync_copy(
                    zb, zdst.at[pl.ds(s * SLICE + k * chunk, chunk)])
            rem = SLICE % chunk
            if rem:
                pltpu.sync_copy(
                    zb.at[pl.ds(0, rem)],
                    zdst.at[pl.ds(s * SLICE + SLICE - rem, rem)])
        if with_deg:
            def oloop(i, carry):
                onesb[pl.ds(i * L, L)] = jnp.ones((L,), jnp.float32)
                return carry
            lax.fori_loop(0, chunk // L, oloop, 0)
        plsc.subcore_barrier()

        def group_body(g, carry):
            for k in range(grp):
                ci = g * grp + k
                sl = k % depth
                nsl = (k + 1) % depth
                s2 = k % 2
                ns2 = (k + 1) % 2

                @pl.when(jnp.logical_and(ci >= depth - 1,
                                         ci < n_chunks + depth - 1))
                def _wait_scatter():
                    pltpu.make_async_copy(
                        valb[nsl], s_sh.at[dstb[nsl]], sc.at[nsl]).wait()
                    if with_deg:
                        pltpu.make_async_copy(
                            onesb, deg_sh.at[dstb[nsl]], dg.at[nsl]).wait()

                @pl.when(ci + 1 < n_chunks)
                def _prefetch():
                    nbase = ebase + (ci + 1) * chunk
                    pltpu.async_copy(src_hbm.at[pl.ds(nbase, chunk)],
                                     srcb[ns2], es.at[ns2])
                    pltpu.async_copy(dst_hbm.at[pl.ds(nbase, chunk)],
                                     dstb[nsl], ed.at[nsl])

                @pl.when(ci < n_chunks)
                def _process():
                    pltpu.make_async_copy(
                        src_hbm.at[pl.ds(ebase + ci * chunk, chunk)],
                        srcb[s2], es.at[s2]).wait()
                    pltpu.make_async_copy(
                        dst_hbm.at[pl.ds(ebase + ci * chunk, chunk)],
                        dstb[sl], ed.at[sl]).wait()

                    sbb, dbb, vbb = srcb[s2], dstb[sl], valb[sl]

                    @plsc.parallel_loop(0, chunk // L, unroll=5)
                    def _gather(i):
                        idx = sbb[pl.ds(i * L, L)]
                        vbb[pl.ds(i * L, L)] = plsc.load_gather(table, [idx])

                    pltpu.async_copy(vbb, s_sh.at[dbb], sc.at[sl], add=True)
                    if with_deg:
                        pltpu.async_copy(onesb, deg_sh.at[dbb],
                                         dg.at[sl], add=True)
            return carry
        lax.fori_loop(0, n_groups, group_body, 0)

        plsc.subcore_barrier()
        off = c * N_PAD + s * SLICE
        pltpu.sync_copy(s_sh.at[pl.ds(s * SLICE, SLICE)],
                        s_out.at[pl.ds(off, SLICE)])
        if with_deg:
            pltpu.sync_copy(deg_sh.at[pl.ds(s * SLICE, SLICE)],
                            deg_out.at[pl.ds(off, SLICE)])

    return pl.kernel(body, out_type=out_type, mesh=_mesh,
                     scratch_types=scratch,
                     compiler_params=pltpu.CompilerParams(
                         needs_layout_passes=False))


_edge_l1 = _make_edge_pass(with_deg=True, depth=4, chunk=800)
_edge_l2 = _make_edge_pass(with_deg=False, depth=5)


def _node_pass1_body(p_ref, s_ref, d_ref, h2_ref, dinv_ref):
    s1 = s_ref[0] + s_ref[1]
    cnt = d_ref[0] + d_ref[1]
    deg = jnp.maximum(cnt, 1.0)
    dinv = 1.0 / deg
    t = s1 * dinv
    h2 = jnp.zeros_like(t)
    for j in range(3):
        h2 = h2 + jnp.maximum(t * p_ref[j] + p_ref[3 + j], 0.0) * p_ref[6 + j]
    h2_ref[...] = h2
    dinv_ref[...] = dinv


_node_pass1 = pl.pallas_call(
    _node_pass1_body,
    out_shape=[jax.ShapeDtypeStruct((ROWS, 128), jnp.float32),
               jax.ShapeDtypeStruct((ROWS, 128), jnp.float32)],
    in_specs=[pl.BlockSpec(memory_space=pltpu.SMEM),
              pl.BlockSpec(memory_space=pltpu.VMEM),
              pl.BlockSpec(memory_space=pltpu.VMEM)],
)


def _node_pass2_body(p_ref, s_ref, dinv_ref, out_ref):
    s2 = s_ref[0] + s_ref[1]
    out_ref[...] = jnp.maximum(s2 * dinv_ref[...] + p_ref[0], 0.0)


_node_pass2 = pl.pallas_call(
    _node_pass2_body,
    out_shape=jax.ShapeDtypeStruct((ROWS, 128), jnp.float32),
    in_specs=[pl.BlockSpec(memory_space=pltpu.SMEM),
              pl.BlockSpec(memory_space=pltpu.VMEM),
              pl.BlockSpec(memory_space=pltpu.VMEM)],
)


@jax.jit
def kernel(feat, edge_index, W1, b1, W2, b2):
    f32 = jnp.float32
    src = edge_index[0]
    dst = edge_index[1]
    vals0 = feat[:, 0].astype(f32)

    s1p, degp = _edge_l1(vals0, src, dst)
    s1 = s1p[:N_PAD] + s1p[N_PAD:]
    cnt = degp[:N_PAD] + degp[N_PAD:]
    deg = jnp.maximum(cnt, 1.0)
    dinv = 1.0 / deg
    t = s1 * dinv
    h2 = jnp.zeros_like(t)
    for j in range(3):
        h2 = h2 + jnp.maximum(t * W1[0, j] + b1[j], 0.0) * W2[j, 0]
    (s2p,) = _edge_l2(h2, src, dst)
    out = jnp.maximum((s2p[:N_PAD] + s2p[N_PAD:]) * dinv + b2[0], 0.0)
    return out[:N][:, None]


# speedup vs baseline: 1.0808x; 1.0808x over previous
import jax
import jax.numpy as jnp
from jax import lax
from jax.experimental import pallas as pl
from jax.experimental.pallas import tpu as pltpu
from jax.experimental.pallas import tpu_sc as plsc

N = 100000
E = 6400000
NC = 2
NS = 16
L = 16
NW = NC * NS
N_PAD = 100352
SLICE = N_PAD // NS
E_PER_TILE = E // NW
CHUNK = 1600
N_CHUNKS = E_PER_TILE // CHUNK
ROWS = N_PAD // 128

_mesh = plsc.VectorSubcoreMesh(
    core_axis_name="c", subcore_axis_name="s", num_cores=NC, num_subcores=NS
)


def _make_edge_pass(with_deg, depth, chunk=CHUNK):
    out_type = [jax.ShapeDtypeStruct((NC * N_PAD,), jnp.float32)]
    scratch = (
        [pltpu.VMEM((N_PAD,), jnp.float32)]
        + [pltpu.VMEM((chunk,), jnp.int32)] * 2
        + [pltpu.VMEM((chunk,), jnp.int32)] * depth
        + [pltpu.VMEM((chunk,), jnp.float32)] * depth
        + [
            pltpu.VMEM_SHARED((N_PAD,), jnp.float32),
            pltpu.SemaphoreType.DMA((2,)),
            pltpu.SemaphoreType.DMA((depth,)),
            pltpu.SemaphoreType.DMA((depth,)),
        ]
    )
    if with_deg:
        out_type.append(jax.ShapeDtypeStruct((NC * N_PAD,), jnp.float32))
        scratch.append(pltpu.VMEM((chunk,), jnp.float32))
        scratch.append(pltpu.VMEM_SHARED((N_PAD,), jnp.float32))
        scratch.append(pltpu.SemaphoreType.DMA((depth,)))

    n_chunks = E_PER_TILE // chunk
    grp = 2 * depth if depth % 2 else depth
    total_iters = n_chunks + depth - 1
    n_groups = (total_iters + grp - 1) // grp

    def body(values_hbm, ei_hbm, s_out, *rest):
        rest = list(rest)
        if with_deg:
            deg_out = rest.pop(0)
        table = rest.pop(0)
        srcb = [rest.pop(0) for _ in range(2)]
        dstb = [rest.pop(0) for _ in range(depth)]
        valb = [rest.pop(0) for _ in range(depth)]
        s_sh, es, ed, sc = rest[:4]
        if with_deg:
            onesb, deg_sh, dg = rest[4:]
        c = lax.axis_index("c")
        s = lax.axis_index("s")
        wid = s * NC + c
        ebase = wid * E_PER_TILE

        pltpu.async_copy(ei_hbm.at[pl.ds(ebase, chunk)], srcb[0],
                         es.at[0])
        pltpu.async_copy(ei_hbm.at[pl.ds(E + ebase, chunk)], dstb[0],
                         ed.at[0])

        pltpu.sync_copy(values_hbm, table.at[pl.ds(0, values_hbm.shape[0])])

        zb = valb[depth - 1]

        def zloop(i, carry):
            zb[pl.ds(i * L, L)] = jnp.zeros((L,), jnp.float32)
            return carry
        lax.fori_loop(0, chunk // L, zloop, 0)
        zdsts = [s_sh] + ([deg_sh] if with_deg else [])
        for zdst in zdsts:
            for k in range(SLICE // chunk):
                pltpu.sync_copy(
                    zb, zdst.at[pl.ds(s * SLICE + k * chunk, chunk)])
            rem = SLICE % chunk
            if rem:
                pltpu.sync_copy(
                    zb.at[pl.ds(0, rem)],
                    zdst.at[pl.ds(s * SLICE + SLICE - rem, rem)])
        if with_deg:
            def oloop(i, carry):
                onesb[pl.ds(i * L, L)] = jnp.ones((L,), jnp.float32)
                return carry
            lax.fori_loop(0, chunk // L, oloop, 0)
        plsc.subcore_barrier()

        def group_body(g, carry):
            for k in range(grp):
                ci = g * grp + k
                sl = k % depth
                nsl = (k + 1) % depth
                s2 = k % 2
                ns2 = (k + 1) % 2

                @pl.when(jnp.logical_and(ci >= depth - 1,
                                         ci < n_chunks + depth - 1))
                def _wait_scatter():
                    pltpu.make_async_copy(
                        valb[nsl], s_sh.at[dstb[nsl]], sc.at[nsl]).wait()
                    if with_deg:
                        pltpu.make_async_copy(
                            onesb, deg_sh.at[dstb[nsl]], dg.at[nsl]).wait()

                @pl.when(ci + 1 < n_chunks)
                def _prefetch():
                    nbase = ebase + (ci + 1) * chunk
                    pltpu.async_copy(ei_hbm.at[pl.ds(nbase, chunk)],
                                     srcb[ns2], es.at[ns2])
                    pltpu.async_copy(ei_hbm.at[pl.ds(E + nbase, chunk)],
                                     dstb[nsl], ed.at[nsl])

                @pl.when(ci < n_chunks)
                def _process():
                    pltpu.make_async_copy(
                        ei_hbm.at[pl.ds(ebase + ci * chunk, chunk)],
                        srcb[s2], es.at[s2]).wait()
                    pltpu.make_async_copy(
                        ei_hbm.at[pl.ds(E + ebase + ci * chunk, chunk)],
                        dstb[sl], ed.at[sl]).wait()

                    sbb, dbb, vbb = srcb[s2], dstb[sl], valb[sl]

                    @plsc.parallel_loop(0, chunk // L, unroll=5)
                    def _gather(i):
                        idx = sbb[pl.ds(i * L, L)]
                        vbb[pl.ds(i * L, L)] = plsc.load_gather(table, [idx])

                    pltpu.async_copy(vbb, s_sh.at[dbb], sc.at[sl], add=True)
                    if with_deg:
                        pltpu.async_copy(onesb, deg_sh.at[dbb],
                                         dg.at[sl], add=True)
            return carry
        lax.fori_loop(0, n_groups, group_body, 0)

        plsc.subcore_barrier()
        off = c * N_PAD + s * SLICE
        pltpu.sync_copy(s_sh.at[pl.ds(s * SLICE, SLICE)],
                        s_out.at[pl.ds(off, SLICE)])
        if with_deg:
            pltpu.sync_copy(deg_sh.at[pl.ds(s * SLICE, SLICE)],
                            deg_out.at[pl.ds(off, SLICE)])

    return pl.kernel(body, out_type=out_type, mesh=_mesh,
                     scratch_types=scratch,
                     compiler_params=pltpu.CompilerParams(
                         needs_layout_passes=False))


_edge_l1 = _make_edge_pass(with_deg=True, depth=4, chunk=800)
_edge_l2 = _make_edge_pass(with_deg=False, depth=5)


def _node_pass1_body(p_ref, s_ref, d_ref, h2_ref, dinv_ref):
    s1 = s_ref[0] + s_ref[1]
    cnt = d_ref[0] + d_ref[1]
    deg = jnp.maximum(cnt, 1.0)
    dinv = 1.0 / deg
    t = s1 * dinv
    h2 = jnp.zeros_like(t)
    for j in range(3):
        h2 = h2 + jnp.maximum(t * p_ref[j] + p_ref[3 + j], 0.0) * p_ref[6 + j]
    h2_ref[...] = h2
    dinv_ref[...] = dinv


_node_pass1 = pl.pallas_call(
    _node_pass1_body,
    out_shape=[jax.ShapeDtypeStruct((ROWS, 128), jnp.float32),
               jax.ShapeDtypeStruct((ROWS, 128), jnp.float32)],
    in_specs=[pl.BlockSpec(memory_space=pltpu.SMEM),
              pl.BlockSpec(memory_space=pltpu.VMEM),
              pl.BlockSpec(memory_space=pltpu.VMEM)],
)


def _node_pass2_body(p_ref, s_ref, dinv_ref, out_ref):
    s2 = s_ref[0] + s_ref[1]
    out_ref[...] = jnp.maximum(s2 * dinv_ref[...] + p_ref[0], 0.0)


_node_pass2 = pl.pallas_call(
    _node_pass2_body,
    out_shape=jax.ShapeDtypeStruct((ROWS, 128), jnp.float32),
    in_specs=[pl.BlockSpec(memory_space=pltpu.SMEM),
              pl.BlockSpec(memory_space=pltpu.VMEM),
              pl.BlockSpec(memory_space=pltpu.VMEM)],
)


@jax.jit
def kernel(feat, edge_index, W1, b1, W2, b2):
    f32 = jnp.float32
    vals0 = feat[:, 0].astype(f32)

    ei_flat = edge_index.reshape(-1)
    s1p, degp = _edge_l1(vals0, ei_flat)
    params1 = jnp.concatenate([
        W1[0].astype(f32), b1.astype(f32), W2[:, 0].astype(f32),
        jnp.zeros((7,), f32),
    ])
    h2, dinv = _node_pass1(params1,
                           s1p.reshape(NC, ROWS, 128),
                           degp.reshape(NC, ROWS, 128))

    (s2p,) = _edge_l2(h2.reshape(-1), ei_flat)
    params2 = jnp.concatenate([b2.astype(f32), jnp.zeros((7,), f32)])
    out = _node_pass2(params2, s2p.reshape(NC, ROWS, 128), dinv)
    return out.reshape(-1)[:N][:, None]
